# baseline (speedup 1.0000x reference)
import functools

import jax
import jax.numpy as jnp
from jax import lax
from jax.experimental import pallas as pl
from jax.experimental.pallas import tpu as pltpu
from jax.experimental.pallas import tpu_sc as plsc

NUM_EMB = 100000
DIM = 128
BATCH = 4096
HIST = 200
TOTAL = BATCH * HIST

NC, NS = 2, 16
NW = NC * NS
PER_W = TOTAL // NW
CHUNK = 128
N_STEPS = PER_W // (2 * CHUNK)

_mesh = plsc.VectorSubcoreMesh(core_axis_name="c", subcore_axis_name="s")


@functools.partial(
    pl.kernel,
    out_type=jax.ShapeDtypeStruct((TOTAL // 2, DIM), jnp.int32),
    mesh=_mesh,
    scratch_types=[
        pltpu.VMEM((PER_W,), jnp.int32),
        pltpu.VMEM((2, CHUNK, DIM), jnp.float32),
        pltpu.VMEM((2, CHUNK // 2, DIM), jnp.int32),
        pltpu.SemaphoreType.DMA,
        pltpu.SemaphoreType.DMA,
    ],
    compiler_params=pltpu.CompilerParams(needs_layout_passes=False),
)
def _sc_gather_cast(table_hbm, idx_hbm, out_hbm, idx_v, rows_v, pk_v,
                    gsem, ssem):
    wid = lax.axis_index("s") * NC + lax.axis_index("c")
    base = wid * PER_W
    pbase = wid * (PER_W // 2)
    pltpu.sync_copy(idx_hbm.at[pl.ds(base, PER_W)], idx_v)

    def pack_chunk(b):
        rbuf = rows_v.at[b]
        pbuf = pk_v.at[b]
        cols = [
            (lax.iota(jnp.int32, 16) + h * 16) for h in range(DIM // 16)
        ]

        def pair(p, carry):
            ra = jnp.full((16,), 2 * p, dtype=jnp.int32)
            rc = jnp.full((16,), 2 * p + 1, dtype=jnp.int32)
            rp = jnp.full((16,), p, dtype=jnp.int32)
            for h in range(DIM // 16):
                a = plsc.load_gather(rbuf, [ra, cols[h]])
                c = plsc.load_gather(rbuf, [rc, cols[h]])
                w = plsc.bitcast(
                    plsc.pack(a, c, format=plsc.PackFormat.INTERLEAVED),
                    jnp.int32,
                )
                plsc.store_scatter(pbuf, [rp, cols[h]], w)
            return carry

        lax.fori_loop(0, CHUNK // 2, pair, 0)

    def body(i, carry):
        off0 = i * 2 * CHUNK
        off1 = off0 + CHUNK
        poff0 = i * CHUNK
        poff1 = poff0 + CHUNK // 2
        g0 = pltpu.async_copy(
            table_hbm.at[idx_v.at[pl.ds(off0, CHUNK)]], rows_v.at[0], gsem
        )
        g1 = pltpu.async_copy(
            table_hbm.at[idx_v.at[pl.ds(off1, CHUNK)]], rows_v.at[1], gsem
        )
        g0.wait()
        pack_chunk(0)
        s0 = pltpu.async_copy(
            pk_v.at[0], out_hbm.at[pl.ds(pbase + poff0, CHUNK // 2)], ssem
        )
        g1.wait()
        pack_chunk(1)
        s1 = pltpu.async_copy(
            pk_v.at[1], out_hbm.at[pl.ds(pbase + poff1, CHUNK // 2)], ssem
        )
        s0.wait()
        s1.wait()
        return carry

    lax.fori_loop(0, N_STEPS, body, 0)


def kernel(x, weight):
    idx_flat = x.reshape(TOTAL)
    pk = _sc_gather_cast(weight, idx_flat)
    bits = lax.bitcast_convert_type(pk, jnp.bfloat16)
    out = jnp.transpose(bits, (0, 2, 1)).reshape(TOTAL, DIM)
    return out.reshape(BATCH, HIST, DIM)


# speedup vs baseline: 1.2298x; 1.2298x over previous
import functools

import jax
import jax.numpy as jnp
from jax import lax
from jax.experimental import pallas as pl
from jax.experimental.pallas import tpu as pltpu
from jax.experimental.pallas import tpu_sc as plsc

NUM_EMB = 100000
DIM = 128
BATCH = 4096
HIST = 200
TOTAL = BATCH * HIST

NC, NS = 2, 16
NW = NC * NS
PER_W = TOTAL // NW
CHUNK = 128
HALF = CHUNK // 2
NBUF = 4
N_CHUNKS = PER_W // CHUNK

_mesh = plsc.VectorSubcoreMesh(core_axis_name="c", subcore_axis_name="s")


@functools.partial(
    pl.kernel,
    out_type=jax.ShapeDtypeStruct((TOTAL // 2, DIM), jnp.int32),
    mesh=_mesh,
    scratch_types=[
        pltpu.VMEM((PER_W,), jnp.int32),
        pltpu.VMEM((NBUF, CHUNK, DIM), jnp.float32),
        pltpu.VMEM((NBUF, HALF, DIM), jnp.int32),
        pltpu.SemaphoreType.DMA,
        pltpu.SemaphoreType.DMA,
    ],
    compiler_params=pltpu.CompilerParams(needs_layout_passes=False),
)
def _sc_gather_cast(table_hbm, idx_hbm, out_hbm, idx_v, rows_v, pk_v,
                    gsem, ssem):
    wid = lax.axis_index("s") * NC + lax.axis_index("c")
    base = wid * PER_W
    pbase = wid * (PER_W // 2)
    pltpu.sync_copy(idx_hbm.at[pl.ds(base, PER_W)], idx_v)

    def pack_chunk(b):
        def group(q, carry):
            r0 = pl.multiple_of(q * 16, 16)
            p0 = pl.multiple_of(q * 8, 8)
            win = rows_v.at[b, pl.ds(r0, 16)]
            pwin = pk_v.at[b, pl.ds(p0, 8)]
            for dp in range(8):
                for h in range(DIM // 16):
                    a = win[2 * dp, pl.ds(h * 16, 16)]
                    c = win[2 * dp + 1, pl.ds(h * 16, 16)]
                    w = plsc.bitcast(
                        plsc.pack(a, c, format=plsc.PackFormat.INTERLEAVED),
                        jnp.int32,
                    )
                    pwin[dp, pl.ds(h * 16, 16)] = w
            return carry

        lax.fori_loop(0, CHUNK // 16, group, 0)

    for b in range(NBUF):
        pltpu.async_copy(
            table_hbm.at[idx_v.at[pl.ds(b * CHUNK, CHUNK)]], rows_v.at[b],
            gsem,
        )

    def body(i, carry):
        b = lax.rem(i, NBUF)
        pltpu.make_async_copy(
            table_hbm.at[idx_v.at[pl.ds(0, CHUNK)]], rows_v.at[0], gsem
        ).wait()
        pack_chunk(b)
        pltpu.async_copy(
            pk_v.at[b], out_hbm.at[pl.ds(pbase + i * HALF, HALF)], ssem
        )

        @pl.when(i >= 1)
        def _drain_store():
            pltpu.make_async_copy(
                pk_v.at[0], out_hbm.at[pl.ds(pbase, HALF)], ssem
            ).wait()

        @pl.when(i + NBUF < N_CHUNKS)
        def _issue_next():
            off = (i + NBUF) * CHUNK
            pltpu.async_copy(
                table_hbm.at[idx_v.at[pl.ds(off, CHUNK)]], rows_v.at[b], gsem
            )

        return carry

    lax.fori_loop(0, N_CHUNKS, body, 0)
    pltpu.make_async_copy(
        pk_v.at[0], out_hbm.at[pl.ds(pbase, HALF)], ssem
    ).wait()


def kernel(x, weight):
    idx_flat = x.reshape(TOTAL)
    pk = _sc_gather_cast(weight, idx_flat)
    bits = lax.bitcast_convert_type(pk, jnp.bfloat16)
    out = jnp.transpose(bits, (0, 2, 1)).reshape(TOTAL, DIM)
    return out.reshape(BATCH, HIST, DIM)


# speedup vs baseline: 1.9636x; 1.5967x over previous
import functools

import jax
import jax.numpy as jnp
from jax import lax
from jax.experimental import pallas as pl
from jax.experimental.pallas import tpu as pltpu
from jax.experimental.pallas import tpu_sc as plsc

NUM_EMB = 100000
DIM = 128
BATCH = 4096
HIST = 200
TOTAL = BATCH * HIST

NC, NS = 2, 16
NW = NC * NS
PER_W = TOTAL // NW
CHUNK = 128
HALF = CHUNK // 2
NBUF = 4
N_CHUNKS = PER_W // CHUNK

_mesh = plsc.VectorSubcoreMesh(core_axis_name="c", subcore_axis_name="s")


@functools.partial(
    pl.kernel,
    out_type=jax.ShapeDtypeStruct((TOTAL // 2, DIM), jnp.int32),
    mesh=_mesh,
    scratch_types=[
        pltpu.VMEM((PER_W,), jnp.int32),
        pltpu.VMEM((NBUF, CHUNK, DIM), jnp.float32),
        pltpu.VMEM((NBUF, HALF, DIM), jnp.int32),
        pltpu.SemaphoreType.DMA,
        pltpu.SemaphoreType.DMA,
    ],
    compiler_params=pltpu.CompilerParams(needs_layout_passes=False),
)
def _sc_gather_cast(table_hbm, idx_hbm, out_hbm, idx_v, rows_v, pk_v,
                    gsem, ssem):
    wid = lax.axis_index("s") * NC + lax.axis_index("c")
    base = wid * PER_W
    pbase = wid * (PER_W // 2)
    pltpu.sync_copy(idx_hbm.at[pl.ds(base, PER_W)], idx_v)

    def pack_chunk(b):
        def group(q, carry):
            r0 = pl.multiple_of(q * 16, 16)
            p0 = pl.multiple_of(q * 8, 8)
            win = rows_v.at[b, pl.ds(r0, 16)]
            pwin = pk_v.at[b, pl.ds(p0, 8)]
            for dp in range(8):
                for h in range(DIM // 16):
                    a = win[2 * dp, pl.ds(h * 16, 16)]
                    c = win[2 * dp + 1, pl.ds(h * 16, 16)]
                    w = plsc.bitcast(
                        plsc.pack(a, c, format=plsc.PackFormat.INTERLEAVED),
                        jnp.int32,
                    )
                    pwin[dp, pl.ds(h * 16, 16)] = w
            return carry

        lax.fori_loop(0, CHUNK // 16, group, 0)

    for b in range(NBUF):
        pltpu.async_copy(
            table_hbm.at[idx_v.at[pl.ds(b * CHUNK, CHUNK)]], rows_v.at[b],
            gsem,
        )

    def body(i, carry):
        b = lax.rem(i, NBUF)
        pltpu.make_async_copy(
            table_hbm.at[idx_v.at[pl.ds(0, CHUNK)]], rows_v.at[0], gsem
        ).wait()
        pack_chunk(b)
        pltpu.async_copy(
            pk_v.at[b], out_hbm.at[pl.ds(pbase + i * HALF, HALF)], ssem
        )

        @pl.when(i >= 1)
        def _drain_store():
            pltpu.make_async_copy(
                pk_v.at[0], out_hbm.at[pl.ds(pbase, HALF)], ssem
            ).wait()

        @pl.when(i + NBUF < N_CHUNKS)
        def _issue_next():
            off = (i + NBUF) * CHUNK
            pltpu.async_copy(
                table_hbm.at[idx_v.at[pl.ds(off, CHUNK)]], rows_v.at[b], gsem
            )

        return carry

    lax.fori_loop(0, N_CHUNKS, body, 0)
    pltpu.make_async_copy(
        pk_v.at[0], out_hbm.at[pl.ds(pbase, HALF)], ssem
    ).wait()


UNPACK_BLK = 4096


def _unpack_body(x_ref, o_ref):
    o_ref[...] = pltpu.bitcast(x_ref[...], jnp.bfloat16)


def _unpack_rows(pk):
    return pl.pallas_call(
        _unpack_body,
        out_shape=jax.ShapeDtypeStruct((TOTAL, DIM), jnp.bfloat16),
        grid=(TOTAL // 2 // UNPACK_BLK,),
        in_specs=[pl.BlockSpec((UNPACK_BLK, DIM), lambda i: (i, 0))],
        out_specs=pl.BlockSpec((2 * UNPACK_BLK, DIM), lambda i: (i, 0)),
    )(pk)


def kernel(x, weight):
    idx_flat = x.reshape(TOTAL)
    pk = _sc_gather_cast(weight, idx_flat)
    out = _unpack_rows(pk)
    return out.reshape(BATCH, HIST, DIM)


# speedup vs baseline: 3.0549x; 1.5557x over previous
import functools

import jax
import jax.numpy as jnp
from jax import lax
from jax.experimental import pallas as pl
from jax.experimental.pallas import tpu as pltpu
from jax.experimental.pallas import tpu_sc as plsc

NUM_EMB = 100000
DIM = 128
BATCH = 4096
HIST = 200
TOTAL = BATCH * HIST

NC, NS = 2, 16
NW = NC * NS
PER_W = TOTAL // NW
CHUNK = 128
HALF = CHUNK // 2
NBUF = 4
N_CHUNKS = PER_W // CHUNK

_mesh = plsc.VectorSubcoreMesh(core_axis_name="c", subcore_axis_name="s")


@functools.partial(
    pl.kernel,
    out_type=jax.ShapeDtypeStruct((TOTAL // 2, DIM), jnp.int32),
    mesh=_mesh,
    scratch_types=[
        pltpu.VMEM((PER_W,), jnp.int32),
        pltpu.VMEM((NBUF, CHUNK, DIM), jnp.float32),
        pltpu.VMEM((NBUF, HALF, DIM), jnp.int32),
        pltpu.SemaphoreType.DMA,
        pltpu.SemaphoreType.DMA,
    ],
    compiler_params=pltpu.CompilerParams(needs_layout_passes=False),
)
def _sc_gather_cast(table_hbm, idx_hbm, out_hbm, idx_v, rows_v, pk_v,
                    gsem, ssem):
    wid = lax.axis_index("s") * NC + lax.axis_index("c")
    base = wid * PER_W
    pbase = wid * (PER_W // 2)
    pltpu.sync_copy(idx_hbm.at[pl.ds(base, PER_W)], idx_v)

    def pack_chunk(b):
        @plsc.parallel_loop(0, CHUNK // 16, unroll=2)
        def _group(q):
            r0 = pl.multiple_of(q * 16, 16)
            p0 = pl.multiple_of(q * 8, 8)
            win = rows_v.at[b, pl.ds(r0, 16)]
            pwin = pk_v.at[b, pl.ds(p0, 8)]
            for dp in range(8):
                for h in range(DIM // 16):
                    a = win[2 * dp, pl.ds(h * 16, 16)]
                    c = win[2 * dp + 1, pl.ds(h * 16, 16)]
                    w = plsc.bitcast(
                        plsc.pack(a, c, format=plsc.PackFormat.INTERLEAVED),
                        jnp.int32,
                    )
                    pwin[dp, pl.ds(h * 16, 16)] = w

    for b in range(NBUF):
        pltpu.async_copy(
            table_hbm.at[idx_v.at[pl.ds(b * CHUNK, CHUNK)]], rows_v.at[b],
            gsem,
        )

    def body(i, carry):
        b = lax.rem(i, NBUF)
        pltpu.make_async_copy(
            table_hbm.at[idx_v.at[pl.ds(0, CHUNK)]], rows_v.at[0], gsem
        ).wait()
        pack_chunk(b)
        pltpu.async_copy(
            pk_v.at[b], out_hbm.at[pl.ds(pbase + i * HALF, HALF)], ssem
        )

        @pl.when(i >= 1)
        def _drain_store():
            pltpu.make_async_copy(
                pk_v.at[0], out_hbm.at[pl.ds(pbase, HALF)], ssem
            ).wait()

        @pl.when(i + NBUF < N_CHUNKS)
        def _issue_next():
            off = (i + NBUF) * CHUNK
            pltpu.async_copy(
                table_hbm.at[idx_v.at[pl.ds(off, CHUNK)]], rows_v.at[b], gsem
            )

        return carry

    lax.fori_loop(0, N_CHUNKS, body, 0)
    pltpu.make_async_copy(
        pk_v.at[0], out_hbm.at[pl.ds(pbase, HALF)], ssem
    ).wait()


UNPACK_BLK = 4096


def _unpack_body(x_ref, o_ref):
    o_ref[...] = pltpu.bitcast(x_ref[...], jnp.bfloat16)


def _unpack_rows(pk):
    return pl.pallas_call(
        _unpack_body,
        out_shape=jax.ShapeDtypeStruct((TOTAL, DIM), jnp.bfloat16),
        grid=(TOTAL // 2 // UNPACK_BLK,),
        in_specs=[pl.BlockSpec((UNPACK_BLK, DIM), lambda i: (i, 0))],
        out_specs=pl.BlockSpec((2 * UNPACK_BLK, DIM), lambda i: (i, 0)),
    )(pk)


def kernel(x, weight):
    idx_flat = x.reshape(TOTAL)
    pk = _sc_gather_cast(weight, idx_flat)
    out = _unpack_rows(pk)
    return out.reshape(BATCH, HIST, DIM)


# speedup vs baseline: 3.1078x; 1.0173x over previous
import functools

import jax
import jax.numpy as jnp
from jax import lax
from jax.experimental import pallas as pl
from jax.experimental.pallas import tpu as pltpu
from jax.experimental.pallas import tpu_sc as plsc

NUM_EMB = 100000
DIM = 128
BATCH = 4096
HIST = 200
TOTAL = BATCH * HIST

NC, NS = 2, 16
NW = NC * NS
N_SEG = 4
SEG = TOTAL // N_SEG
PER_W = SEG // NW
CHUNK = 128
HALF = CHUNK // 2
NBUF = 4
N_CHUNKS = PER_W // CHUNK

_mesh = plsc.VectorSubcoreMesh(core_axis_name="c", subcore_axis_name="s")


@functools.partial(
    pl.kernel,
    out_type=jax.ShapeDtypeStruct((SEG // 2, DIM), jnp.int32),
    mesh=_mesh,
    scratch_types=[
        pltpu.VMEM((PER_W,), jnp.int32),
        pltpu.VMEM((NBUF, CHUNK, DIM), jnp.float32),
        pltpu.VMEM((NBUF, HALF, DIM), jnp.int32),
        pltpu.SemaphoreType.DMA,
        pltpu.SemaphoreType.DMA,
    ],
    compiler_params=pltpu.CompilerParams(needs_layout_passes=False),
)
def _sc_gather_cast(table_hbm, idx_hbm, out_hbm, idx_v, rows_v, pk_v,
                    gsem, ssem):
    wid = lax.axis_index("s") * NC + lax.axis_index("c")
    base = wid * PER_W
    pbase = wid * (PER_W // 2)
    pltpu.sync_copy(idx_hbm.at[pl.ds(base, PER_W)], idx_v)

    def pack_chunk(b):
        @plsc.parallel_loop(0, CHUNK // 16, unroll=2)
        def _group(q):
            r0 = pl.multiple_of(q * 16, 16)
            p0 = pl.multiple_of(q * 8, 8)
            win = rows_v.at[b, pl.ds(r0, 16)]
            pwin = pk_v.at[b, pl.ds(p0, 8)]
            for dp in range(8):
                for h in range(DIM // 16):
                    a = win[2 * dp, pl.ds(h * 16, 16)]
                    c = win[2 * dp + 1, pl.ds(h * 16, 16)]
                    w = plsc.bitcast(
                        plsc.pack(a, c, format=plsc.PackFormat.INTERLEAVED),
                        jnp.int32,
                    )
                    pwin[dp, pl.ds(h * 16, 16)] = w

    for b in range(NBUF):
        pltpu.async_copy(
            table_hbm.at[idx_v.at[pl.ds(b * CHUNK, CHUNK)]], rows_v.at[b],
            gsem,
        )

    def body(i, carry):
        b = lax.rem(i, NBUF)
        pltpu.make_async_copy(
            table_hbm.at[idx_v.at[pl.ds(0, CHUNK)]], rows_v.at[0], gsem
        ).wait()
        pack_chunk(b)
        pltpu.async_copy(
            pk_v.at[b], out_hbm.at[pl.ds(pbase + i * HALF, HALF)], ssem
        )

        @pl.when(i >= 1)
        def _drain_store():
            pltpu.make_async_copy(
                pk_v.at[0], out_hbm.at[pl.ds(pbase, HALF)], ssem
            ).wait()

        @pl.when(i + NBUF < N_CHUNKS)
        def _issue_next():
            off = (i + NBUF) * CHUNK
            pltpu.async_copy(
                table_hbm.at[idx_v.at[pl.ds(off, CHUNK)]], rows_v.at[b], gsem
            )

        return carry

    lax.fori_loop(0, N_CHUNKS, body, 0)
    pltpu.make_async_copy(
        pk_v.at[0], out_hbm.at[pl.ds(pbase, HALF)], ssem
    ).wait()


UNPACK_BLK = 4096
SEG_BLKS = SEG // 2 // UNPACK_BLK


def _unpack_first_body(x_ref, o_ref):
    o_ref[...] = pltpu.bitcast(x_ref[...], jnp.bfloat16)


def _unpack_seg_body(_, x_ref, o_ref):
    o_ref[...] = pltpu.bitcast(x_ref[...], jnp.bfloat16)


def _unpack_segment(seg, buf, pk):
    out_spec = pl.BlockSpec(
        (2 * UNPACK_BLK, DIM), lambda i, s=seg: (s * SEG_BLKS + i, 0)
    )
    in_spec = pl.BlockSpec((UNPACK_BLK, DIM), lambda i: (i, 0))
    if buf is None:
        return pl.pallas_call(
            _unpack_first_body,
            out_shape=jax.ShapeDtypeStruct((TOTAL, DIM), jnp.bfloat16),
            grid=(SEG_BLKS,),
            in_specs=[in_spec],
            out_specs=out_spec,
        )(pk)
    return pl.pallas_call(
        _unpack_seg_body,
        out_shape=jax.ShapeDtypeStruct((TOTAL, DIM), jnp.bfloat16),
        grid=(SEG_BLKS,),
        in_specs=[pl.BlockSpec(memory_space=pl.ANY), in_spec],
        out_specs=out_spec,
        input_output_aliases={0: 0},
    )(buf, pk)


def kernel(x, weight):
    idx_flat = x.reshape(TOTAL)
    buf = None
    for s in range(N_SEG):
        pk = _sc_gather_cast(weight, idx_flat[s * SEG:(s + 1) * SEG])
        buf = _unpack_segment(s, buf, pk)
    return buf.reshape(BATCH, HIST, DIM)


# speedup vs baseline: 4.7271x; 1.5210x over previous
import functools

import jax
import jax.numpy as jnp
from jax import lax
from jax.experimental import pallas as pl
from jax.experimental.pallas import tpu as pltpu
from jax.experimental.pallas import tpu_sc as plsc

NUM_EMB = 100000
DIM = 128
BATCH = 4096
HIST = 200
TOTAL = BATCH * HIST

NC, NS = 2, 16
NW = NC * NS
PER_W = TOTAL // NW
CHUNK = 128
HALF = CHUNK // 2
NBUF = 4
N_CHUNKS = PER_W // CHUNK

_mesh = plsc.VectorSubcoreMesh(core_axis_name="c", subcore_axis_name="s")


@functools.partial(
    pl.kernel,
    out_type=jax.ShapeDtypeStruct((TOTAL, DIM), jnp.bfloat16),
    mesh=_mesh,
    scratch_types=[
        pltpu.VMEM((PER_W,), jnp.int32),
        pltpu.VMEM((NBUF, CHUNK, DIM), jnp.float32),
        pltpu.VMEM((NBUF, HALF, DIM), jnp.int32),
        pltpu.SemaphoreType.DMA,
        pltpu.SemaphoreType.DMA,
    ],
    compiler_params=pltpu.CompilerParams(needs_layout_passes=False),
)
def _sc_gather_cast(table_hbm, idx_hbm, out_hbm, idx_v, rows_v, pk_v,
                    gsem, ssem):
    wid = lax.axis_index("s") * NC + lax.axis_index("c")
    base = wid * PER_W
    pltpu.sync_copy(idx_hbm.at[pl.ds(base, PER_W)], idx_v)

    def pack_chunk(b):
        @plsc.parallel_loop(0, CHUNK // 16, unroll=2)
        def _group(q):
            r0 = pl.multiple_of(q * 16, 16)
            p0 = pl.multiple_of(q * 8, 8)
            win = rows_v.at[b, pl.ds(r0, 16)]
            pwin = pk_v.at[b, pl.ds(p0, 8)]
            for dp in range(8):
                for h in range(DIM // 16):
                    a = win[2 * dp, pl.ds(h * 16, 16)]
                    c = win[2 * dp + 1, pl.ds(h * 16, 16)]
                    w = plsc.bitcast(
                        plsc.pack(a, c, format=plsc.PackFormat.INTERLEAVED),
                        jnp.int32,
                    )
                    pwin[dp, pl.ds(h * 16, 16)] = w

    for b in range(NBUF):
        pltpu.async_copy(
            table_hbm.at[idx_v.at[pl.ds(b * CHUNK, CHUNK)]], rows_v.at[b],
            gsem,
        )

    def body(i, carry):
        b = lax.rem(i, NBUF)
        pltpu.make_async_copy(
            table_hbm.at[idx_v.at[pl.ds(0, CHUNK)]], rows_v.at[0], gsem
        ).wait()
        pack_chunk(b)
        pltpu.async_copy(
            pk_v.at[b].bitcast(jnp.bfloat16),
            out_hbm.at[pl.ds(base + i * CHUNK, CHUNK)],
            ssem,
        )

        @pl.when(i >= 1)
        def _drain_store():
            pltpu.make_async_copy(
                pk_v.at[0].bitcast(jnp.bfloat16),
                out_hbm.at[pl.ds(base, CHUNK)],
                ssem,
            ).wait()

        @pl.when(i + NBUF < N_CHUNKS)
        def _issue_next():
            off = (i + NBUF) * CHUNK
            pltpu.async_copy(
                table_hbm.at[idx_v.at[pl.ds(off, CHUNK)]], rows_v.at[b], gsem
            )

        return carry

    lax.fori_loop(0, N_CHUNKS, body, 0)
    pltpu.make_async_copy(
        pk_v.at[0].bitcast(jnp.bfloat16), out_hbm.at[pl.ds(base, CHUNK)], ssem
    ).wait()


def kernel(x, weight):
    idx_flat = x.reshape(TOTAL)
    out = _sc_gather_cast(weight, idx_flat)
    return out.reshape(BATCH, HIST, DIM)


# speedup vs baseline: 4.7293x; 1.0005x over previous
import functools

import jax
import jax.numpy as jnp
from jax import lax
from jax.experimental import pallas as pl
from jax.experimental.pallas import tpu as pltpu
from jax.experimental.pallas import tpu_sc as plsc

NUM_EMB = 100000
DIM = 128
BATCH = 4096
HIST = 200
TOTAL = BATCH * HIST

NC, NS = 2, 16
NW = NC * NS
PER_W = TOTAL // NW
CHUNK = 128
HALF = CHUNK // 2
NBUF = 4
N_CHUNKS = PER_W // CHUNK

_mesh = plsc.VectorSubcoreMesh(core_axis_name="c", subcore_axis_name="s")


@functools.partial(
    pl.kernel,
    out_type=jax.ShapeDtypeStruct((TOTAL, DIM), jnp.bfloat16),
    mesh=_mesh,
    scratch_types=[
        pltpu.VMEM((PER_W,), jnp.int32),
        pltpu.VMEM((NBUF, CHUNK, DIM), jnp.float32),
        pltpu.VMEM((NBUF, HALF, DIM), jnp.int32),
        pltpu.SemaphoreType.DMA,
        pltpu.SemaphoreType.DMA,
    ],
    compiler_params=pltpu.CompilerParams(needs_layout_passes=False),
)
def _sc_gather_cast(table_hbm, idx_hbm, out_hbm, idx_v, rows_v, pk_v,
                    gsem, ssem):
    wid = lax.axis_index("s") * NC + lax.axis_index("c")
    base = wid * PER_W
    pltpu.sync_copy(idx_hbm.at[pl.ds(base, PER_W)], idx_v)

    def pack_chunk(b):
        @plsc.parallel_loop(0, CHUNK // 16, unroll=4)
        def _group(q):
            r0 = pl.multiple_of(q * 16, 16)
            p0 = pl.multiple_of(q * 8, 8)
            win = rows_v.at[b, pl.ds(r0, 16)]
            pwin = pk_v.at[b, pl.ds(p0, 8)]
            for dp in range(8):
                for h in range(DIM // 16):
                    a = win[2 * dp, pl.ds(h * 16, 16)]
                    c = win[2 * dp + 1, pl.ds(h * 16, 16)]
                    w = plsc.bitcast(
                        plsc.pack(a, c, format=plsc.PackFormat.INTERLEAVED),
                        jnp.int32,
                    )
                    pwin[dp, pl.ds(h * 16, 16)] = w

    for b in range(NBUF):
        pltpu.async_copy(
            table_hbm.at[idx_v.at[pl.ds(b * CHUNK, CHUNK)]], rows_v.at[b],
            gsem,
        )

    def body(i, carry):
        b = lax.rem(i, NBUF)
        pltpu.make_async_copy(
            table_hbm.at[idx_v.at[pl.ds(0, CHUNK)]], rows_v.at[0], gsem
        ).wait()
        pack_chunk(b)
        pltpu.async_copy(
            pk_v.at[b].bitcast(jnp.bfloat16),
            out_hbm.at[pl.ds(base + i * CHUNK, CHUNK)],
            ssem,
        )

        @pl.when(i >= 1)
        def _drain_store():
            pltpu.make_async_copy(
                pk_v.at[0].bitcast(jnp.bfloat16),
                out_hbm.at[pl.ds(base, CHUNK)],
                ssem,
            ).wait()

        @pl.when(i + NBUF < N_CHUNKS)
        def _issue_next():
            off = (i + NBUF) * CHUNK
            pltpu.async_copy(
                table_hbm.at[idx_v.at[pl.ds(off, CHUNK)]], rows_v.at[b], gsem
            )

        return carry

    lax.fori_loop(0, N_CHUNKS, body, 0)
    pltpu.make_async_copy(
        pk_v.at[0].bitcast(jnp.bfloat16), out_hbm.at[pl.ds(base, CHUNK)], ssem
    ).wait()


def kernel(x, weight):
    idx_flat = x.reshape(TOTAL)
    out = _sc_gather_cast(weight, idx_flat)
    return out.reshape(BATCH, HIST, DIM)


# speedup vs baseline: 4.7294x; 1.0000x over previous
import functools

import jax
import jax.numpy as jnp
from jax import lax
from jax.experimental import pallas as pl
from jax.experimental.pallas import tpu as pltpu
from jax.experimental.pallas import tpu_sc as plsc

NUM_EMB = 100000
DIM = 128
BATCH = 4096
HIST = 200
TOTAL = BATCH * HIST

NC, NS = 2, 16
NW = NC * NS
PER_W = TOTAL // NW
CHUNK = 128
HALF = CHUNK // 2
NBUF = 4
N_CHUNKS = PER_W // CHUNK

_mesh = plsc.VectorSubcoreMesh(core_axis_name="c", subcore_axis_name="s")


@functools.partial(
    pl.kernel,
    out_type=jax.ShapeDtypeStruct((TOTAL, DIM), jnp.bfloat16),
    mesh=_mesh,
    scratch_types=[
        pltpu.VMEM((PER_W,), jnp.int32),
        pltpu.VMEM((NBUF, CHUNK, DIM), jnp.float32),
        pltpu.VMEM((NBUF, HALF, DIM), jnp.int32),
        pltpu.SemaphoreType.DMA,
        pltpu.SemaphoreType.DMA,
    ],
    compiler_params=pltpu.CompilerParams(needs_layout_passes=False),
)
def _sc_gather_cast(table_hbm, idx_hbm, out_hbm, idx_v, rows_v, pk_v,
                    gsem, ssem):
    wid = lax.axis_index("s") * NC + lax.axis_index("c")
    base = wid * PER_W
    pltpu.sync_copy(idx_hbm.at[pl.ds(base, PER_W)], idx_v)

    def pack_chunk(b):
        @plsc.parallel_loop(0, CHUNK // 16, unroll=2)
        def _group(q):
            r0 = pl.multiple_of(q * 16, 16)
            p0 = pl.multiple_of(q * 8, 8)
            win = rows_v.at[b, pl.ds(r0, 16)]
            pwin = pk_v.at[b, pl.ds(p0, 8)]
            for dp in range(8):
                for h in range(DIM // 16):
                    a = win[2 * dp, pl.ds(h * 16, 16)]
                    c = win[2 * dp + 1, pl.ds(h * 16, 16)]
                    w = plsc.bitcast(
                        plsc.pack(a, c, format=plsc.PackFormat.INTERLEAVED),
                        jnp.int32,
                    )
                    pwin[dp, pl.ds(h * 16, 16)] = w

    for b in range(NBUF):
        pltpu.async_copy(
            table_hbm.at[idx_v.at[pl.ds(b * CHUNK, CHUNK)]], rows_v.at[b],
            gsem,
        )

    def body(i, carry):
        b = lax.rem(i, NBUF)
        pltpu.make_async_copy(
            table_hbm.at[idx_v.at[pl.ds(0, CHUNK)]], rows_v.at[0], gsem
        ).wait()
        pack_chunk(b)
        pltpu.async_copy(
            pk_v.at[b].bitcast(jnp.bfloat16),
            out_hbm.at[pl.ds(base + i * CHUNK, CHUNK)],
            ssem,
        )

        @pl.when(i >= 1)
        def _drain_store():
            pltpu.make_async_copy(
                pk_v.at[0].bitcast(jnp.bfloat16),
                out_hbm.at[pl.ds(base, CHUNK)],
                ssem,
            ).wait()

        @pl.when(i + NBUF < N_CHUNKS)
        def _issue_next():
            off = (i + NBUF) * CHUNK
            pltpu.async_copy(
                table_hbm.at[idx_v.at[pl.ds(off, CHUNK)]], rows_v.at[b], gsem
            )

        return carry

    lax.fori_loop(0, N_CHUNKS, body, 0)
    pltpu.make_async_copy(
        pk_v.at[0].bitcast(jnp.bfloat16), out_hbm.at[pl.ds(base, CHUNK)], ssem
    ).wait()


def kernel(x, weight):
    idx_flat = x.reshape(TOTAL)
    out = _sc_gather_cast(weight, idx_flat)
    return out.reshape(BATCH, HIST, DIM)
